# baseline (speedup 1.0000x reference)
import functools

import jax
import jax.numpy as jnp
from jax import lax
from jax.experimental import pallas as pl
from jax.experimental.pallas import tpu as pltpu
from jax.experimental.pallas import tpu_sc as plsc

N = 10000
NT = 32
NR = 320
NPAD = NT * NR
E = 320000
D = 128
L = 16
NC = 2

EPW = E // NT
DC = 2000
CE = 4000
NCHUNK = E // CE
GB = 128
EBCAP = 4096

def _sc_deg_body(dst_hbm, ew_hbm, out_hbm, dstb, ewb, dacc0, dacc1):
    c = lax.axis_index("c")
    s = lax.axis_index("s")
    wid = s * NC + c
    daccs = (dacc0, dacc1)
    zero = jnp.zeros((L,), jnp.float32)

    def zbody(i, _):
        dacc0[pl.ds(i * L, L)] = zero
        dacc1[pl.ds(i * L, L)] = zero
        return 0

    lax.fori_loop(0, NPAD // L, zbody, 0)

    for t in range(2):
        dacc = daccs[t]

        def cbody(k, _):
            base = t * E + wid * EPW + k * DC
            pltpu.sync_copy(dst_hbm.at[pl.ds(base, DC)], dstb)
            pltpu.sync_copy(ew_hbm.at[pl.ds(base, DC)], ewb)

            def vbody(i, _):
                d = dstb[pl.ds(i * L, L)]
                w = jnp.maximum(ewb[pl.ds(i * L, L)], 0.0)
                plsc.addupdate_scatter(dacc, [d], w)
                return 0

            lax.fori_loop(0, DC // L, vbody, 0)
            return 0

        lax.fori_loop(0, EPW // DC, cbody, 0)

    pltpu.sync_copy(dacc0, out_hbm.at[pl.ds((wid * 2) * NPAD, NPAD)])
    pltpu.sync_copy(dacc1, out_hbm.at[pl.ds((wid * 2 + 1) * NPAD, NPAD)])


def _prep_body(x_ref, w_ref, dp_ref, xl_ref, dinv_ref):
    x = x_ref[0]
    xl = jnp.dot(x, w_ref[...], preferred_element_type=jnp.float32)
    deg2d = jnp.sum(dp_ref[0], axis=0) + 2.0
    dinv_ref[0] = lax.rsqrt(deg2d)
    xl_ref[0] = xl


def _prep(x_st, W, dpT):
    return pl.pallas_call(
        _prep_body,
        grid=(2,),
        in_specs=[
            pl.BlockSpec((1, NPAD, D), lambda t: (t, 0, 0)),
            pl.BlockSpec((D, D), lambda t: (0, 0)),
            pl.BlockSpec((1, NT, 8, 1280), lambda t: (t, 0, 0, 0)),
        ],
        out_specs=[
            pl.BlockSpec((1, NPAD, D), lambda t: (t, 0, 0)),
            pl.BlockSpec((1, 8, 1280), lambda t: (t, 0, 0)),
        ],
        out_shape=[
            jax.ShapeDtypeStruct((2, NPAD, D), jnp.float32),
            jax.ShapeDtypeStruct((2, 8, 1280), jnp.float32),
        ],
    )(x_st, W, dpT)


def _sc_conv_body(src_hbm, dst_hbm, ew_hbm, dinv_hbm, xl_hbm, b_hbm,
                  out_hbm, dstb, srcb, ewb, es, ed, en, dinv_v, acc, rows,
                  bias, gsem):
    c = lax.axis_index("c")
    s = lax.axis_index("s")
    wid = s * NC + c
    lo = wid * NR
    pltpu.sync_copy(b_hbm, bias)

    zi = jnp.zeros((L,), jnp.int32)

    def zbody(i, _):
        es[pl.ds(i * L, L)] = zi
        return 0

    lax.fori_loop(0, EBCAP // L, zbody, 0)

    for t in range(2):
        pltpu.sync_copy(dinv_hbm.at[pl.ds(t * NPAD, NPAD)],
                        dinv_v.at[pl.ds(0, NPAD)])
        pltpu.sync_copy(xl_hbm.at[pl.ds(t * NPAD + lo, NR)], acc)

        def sbody(r, _):
            dv = dinv_v[pl.ds(lo + r, L)][0]
            f = jnp.full((L,), 2.0 * dv * dv, jnp.float32)
            for kk in range(D // L):
                sl = pl.ds(kk * L, L)
                acc[r, sl] = acc[r, sl] * f
            return 0

        lax.fori_loop(0, NR, sbody, 0)

        def cbody(k, _):
            base = t * E + k * CE
            pltpu.sync_copy(dst_hbm.at[pl.ds(base, CE)], dstb)
            pltpu.sync_copy(src_hbm.at[pl.ds(base, CE)], srcb)
            pltpu.sync_copy(ew_hbm.at[pl.ds(base, CE)], ewb)

            def vbody(i, cnt):
                d = dstb[pl.ds(i * L, L)]
                m = (d >= lo) & (d < lo + NR)
                nm = jnp.sum(m.astype(jnp.int32))

                @pl.when(nm > 0)
                def _():
                    sv = srcb[pl.ds(i * L, L)]
                    wv = jnp.maximum(ewb[pl.ds(i * L, L)], 0.0)
                    nv = (plsc.load_gather(dinv_v, [sv]) * wv
                          * plsc.load_gather(dinv_v, [d]))
                    plsc.store_compressed(es.at[pl.ds(cnt, L)],
                                          sv + t * NPAD, mask=m)
                    plsc.store_compressed(ed.at[pl.ds(cnt, L)], d - lo, mask=m)
                    plsc.store_compressed(en.at[pl.ds(cnt, L)], nv, mask=m)

                return cnt + nm

            cnt = lax.fori_loop(0, CE // L, vbody, jnp.int32(0))
            nb = lax.shift_right_logical(cnt + (GB - 1), 7)

            def bbody(b, _):
                pltpu.async_copy(xl_hbm.at[es.at[pl.ds(b * GB, GB)]], rows,
                                 gsem).wait()

                def ebody(e, _):
                    eg = b * GB + e

                    @pl.when(eg < cnt)
                    def _():
                        dl = ed[pl.ds(eg, L)][0]
                        spl = jnp.full((L,), en[pl.ds(eg, L)][0], jnp.float32)
                        for kk in range(D // L):
                            sl = pl.ds(kk * L, L)
                            acc[dl, sl] = jnp.maximum(acc[dl, sl],
                                                      rows[e, sl] * spl)

                    return 0

                lax.fori_loop(0, GB, ebody, 0)
                return 0

            lax.fori_loop(0, nb, bbody, 0)
            return 0

        lax.fori_loop(0, NCHUNK, cbody, 0)

        def abody(r, _):
            for kk in range(D // L):
                sl = pl.ds(kk * L, L)
                acc[r, sl] = acc[r, sl] + bias[sl]
            return 0

        lax.fori_loop(0, NR, abody, 0)
        pltpu.sync_copy(acc, out_hbm.at[pl.ds(t * NPAD + lo, NR)])


@functools.cache
def _sc_kernels():
    mesh = plsc.VectorSubcoreMesh(core_axis_name="c", subcore_axis_name="s",
                                  num_cores=NC)
    sc_deg = pl.kernel(
        _sc_deg_body,
        out_type=jax.ShapeDtypeStruct((NT * 2 * NPAD,), jnp.float32),
        scratch_types=[
            pltpu.VMEM((DC,), jnp.int32),
            pltpu.VMEM((DC,), jnp.float32),
            pltpu.VMEM((NPAD,), jnp.float32),
            pltpu.VMEM((NPAD,), jnp.float32),
        ],
        mesh=mesh,
        compiler_params=pltpu.CompilerParams(needs_layout_passes=False),
    )
    sc_conv = pl.kernel(
        _sc_conv_body,
        out_type=jax.ShapeDtypeStruct((2 * NPAD, D), jnp.float32),
        scratch_types=[
            pltpu.VMEM((CE,), jnp.int32),
            pltpu.VMEM((CE,), jnp.int32),
            pltpu.VMEM((CE,), jnp.float32),
            pltpu.VMEM((EBCAP,), jnp.int32),
            pltpu.VMEM((EBCAP,), jnp.int32),
            pltpu.VMEM((EBCAP,), jnp.float32),
            pltpu.VMEM((NPAD + L,), jnp.float32),
            pltpu.VMEM((NR, D), jnp.float32),
            pltpu.VMEM((GB, D), jnp.float32),
            pltpu.VMEM((D,), jnp.float32),
            pltpu.SemaphoreType.DMA,
        ],
        mesh=mesh,
        compiler_params=pltpu.CompilerParams(needs_layout_passes=False),
    )
    return sc_deg, sc_conv


def kernel(x1, edge_index1, edge_weight1, x2, edge_index2, edge_weight2,
           W1, b1, W2, b2):
    _sc_deg, _sc_conv = _sc_kernels()
    src = jnp.concatenate([edge_index1[0], edge_index2[0]]).astype(jnp.int32)
    dst = jnp.concatenate([edge_index1[1], edge_index2[1]]).astype(jnp.int32)
    ew = jnp.concatenate([edge_weight1, edge_weight2]).astype(jnp.float32)
    x = jnp.zeros((2, NPAD, D), jnp.float32)
    x = x.at[:, :N].set(jnp.stack([x1, x2]))

    deg_parts = _sc_deg(dst, ew).reshape(NT, 2, NPAD)
    dpT = deg_parts.transpose(1, 0, 2).reshape(2, NT, 8, 1280)

    xl, dinv3 = _prep(x, W1, dpT)
    dinv = dinv3.reshape(2 * NPAD)
    h = _sc_conv(src, dst, ew, dinv, xl.reshape(2 * NPAD, D), b1)

    xl2, _ = _prep(h.reshape(2, NPAD, D), W2, dpT)
    g = _sc_conv(src, dst, ew, dinv, xl2.reshape(2 * NPAD, D), b2)
    g = g.reshape(2, NPAD, D)
    return (g[0, :N], g[1, :N])
